# Initial kernel scaffold; baseline (speedup 1.0000x reference)
import jax
import jax.numpy as jnp
from jax.experimental import pallas as pl


def kernel(x, edge_index, W1, b1, W2, b2):
    raise NotImplementedError("write your pallas kernel here")


# speedup vs baseline: 10.2508x; 10.2508x over previous
import functools

import jax
import jax.numpy as jnp
from jax import lax
from jax.experimental import pallas as pl
from jax.experimental.pallas import tpu as pltpu
from jax.experimental.pallas import tpu_sc as plsc

N = 10000
D = 128
NPAD = 10240
E = 320000
NC, NS, L = 2, 16, 16
NW = NC * NS
CH = 128
EPT = 10112
K = EPT // CH
EPAD = EPT * NW
RPS = NPAD // NS

_mesh = plsc.VectorSubcoreMesh(core_axis_name="c", subcore_axis_name="s")


@functools.partial(
    pl.kernel,
    out_type=jax.ShapeDtypeStruct((NC, NPAD, D), jnp.float32),
    mesh=_mesh,
    scratch_types=[
        pltpu.VMEM_SHARED((NPAD, D), jnp.float32),
        pltpu.VMEM((CH, D), jnp.float32),
        pltpu.VMEM((CH,), jnp.int32),
        pltpu.VMEM((64, D), jnp.float32),
    ],
)
def _deg_kernel(dst_hbm, hist_hbm, shared_h, ones_v, idx_v, zbuf):
    cid = lax.axis_index("c")
    sid = lax.axis_index("s")
    wid = sid * NC + cid

    def fill(i, _):
        for j in range(D // L):
            zbuf[i, pl.ds(j * L, L)] = jnp.zeros((L,), jnp.float32)
        return 0

    lax.fori_loop(0, 64, fill, 0)

    def fill1(i, _):
        for j in range(D // L):
            ones_v[i, pl.ds(j * L, L)] = jnp.ones((L,), jnp.float32)
        return 0

    lax.fori_loop(0, CH, fill1, 0)
    for j in range(RPS // 64):
        pltpu.sync_copy(zbuf, shared_h.at[pl.ds(sid * RPS + j * 64, 64)])
    plsc.subcore_barrier()

    base = wid * EPT

    def step(k, _):
        pltpu.sync_copy(dst_hbm.at[pl.ds(base + k * CH, CH)], idx_v)
        pltpu.sync_copy(ones_v, shared_h.at[idx_v], add=True)
        return 0

    lax.fori_loop(0, K, step, 0)
    plsc.subcore_barrier()
    pltpu.sync_copy(
        shared_h.at[pl.ds(sid * RPS, RPS)],
        hist_hbm.at[cid, pl.ds(sid * RPS, RPS)],
    )


@functools.partial(
    pl.kernel,
    out_type=jax.ShapeDtypeStruct((NC, NPAD, D), jnp.float32),
    mesh=_mesh,
    scratch_types=[
        pltpu.VMEM_SHARED((NPAD, D), jnp.float32),
        pltpu.VMEM((CH, D), jnp.float32),
        pltpu.VMEM((CH,), jnp.int32),
        pltpu.VMEM((CH,), jnp.int32),
        pltpu.VMEM((64, D), jnp.float32),
        pltpu.SemaphoreType.DMA,
    ],
)
def _edge_kernel(y_hbm, src_hbm, dst_hbm, z_hbm,
                 shared_z, rows_v, sidx, didx, zbuf, sem):
    cid = lax.axis_index("c")
    sid = lax.axis_index("s")
    wid = sid * NC + cid

    def fill(i, _):
        for j in range(D // L):
            zbuf[i, pl.ds(j * L, L)] = jnp.zeros((L,), jnp.float32)
        return 0

    lax.fori_loop(0, 64, fill, 0)
    for j in range(RPS // 64):
        pltpu.sync_copy(zbuf, shared_z.at[pl.ds(sid * RPS + j * 64, 64)])
    plsc.subcore_barrier()

    base = wid * EPT

    def step(k, _):
        pltpu.sync_copy(src_hbm.at[pl.ds(base + k * CH, CH)], sidx)
        pltpu.sync_copy(dst_hbm.at[pl.ds(base + k * CH, CH)], didx)
        pltpu.async_copy(y_hbm.at[sidx], rows_v, sem).wait()
        pltpu.sync_copy(rows_v, shared_z.at[didx], add=True)
        return 0

    lax.fori_loop(0, K, step, 0)
    plsc.subcore_barrier()
    pltpu.sync_copy(
        shared_z.at[pl.ds(sid * RPS, RPS)],
        z_hbm.at[cid, pl.ds(sid * RPS, RPS)],
    )


BR = 1024


def _mm1_body(hist_ref, x_ref, w_ref, y_ref, dinv_ref):
    deg = hist_ref[0][:, 0:1] + hist_ref[1][:, 0:1] + 1.0
    dinv = lax.rsqrt(deg)
    xw = jnp.dot(x_ref[...], w_ref[...], preferred_element_type=jnp.float32)
    y_ref[...] = xw * dinv
    dinv_ref[...] = jnp.broadcast_to(dinv, (BR, D))


def _mm1(hist, xp, W1):
    return pl.pallas_call(
        _mm1_body,
        grid=(NPAD // BR,),
        in_specs=[
            pl.BlockSpec((NC, BR, D), lambda i: (0, i, 0)),
            pl.BlockSpec((BR, D), lambda i: (i, 0)),
            pl.BlockSpec((D, D), lambda i: (0, 0)),
        ],
        out_specs=[
            pl.BlockSpec((BR, D), lambda i: (i, 0)),
            pl.BlockSpec((BR, D), lambda i: (i, 0)),
        ],
        out_shape=[
            jax.ShapeDtypeStruct((NPAD, D), jnp.float32),
            jax.ShapeDtypeStruct((NPAD, D), jnp.float32),
        ],
    )(hist, xp, W1)


def _mm2_body(z_ref, y_ref, dinv_ref, b_ref, w_ref, o_ref):
    h = (z_ref[0] + z_ref[1] + y_ref[...]) * dinv_ref[...] + b_ref[...]
    h = jnp.maximum(h, 0.0)
    o_ref[...] = (
        jnp.dot(h, w_ref[...], preferred_element_type=jnp.float32)
        * dinv_ref[...]
    )


def _mm2(z, y, dinv, b, W2):
    return pl.pallas_call(
        _mm2_body,
        grid=(NPAD // BR,),
        in_specs=[
            pl.BlockSpec((NC, BR, D), lambda i: (0, i, 0)),
            pl.BlockSpec((BR, D), lambda i: (i, 0)),
            pl.BlockSpec((BR, D), lambda i: (i, 0)),
            pl.BlockSpec((1, D), lambda i: (0, 0)),
            pl.BlockSpec((D, D), lambda i: (0, 0)),
        ],
        out_specs=pl.BlockSpec((BR, D), lambda i: (i, 0)),
        out_shape=jax.ShapeDtypeStruct((NPAD, D), jnp.float32),
    )(z, y, dinv, b, W2)


def _fin_body(z_ref, y_ref, dinv_ref, b_ref, o_ref):
    o_ref[...] = (z_ref[0] + z_ref[1] + y_ref[...]) * dinv_ref[...] + b_ref[...]


def _fin(z, y, dinv, b):
    return pl.pallas_call(
        _fin_body,
        grid=(NPAD // BR,),
        in_specs=[
            pl.BlockSpec((NC, BR, D), lambda i: (0, i, 0)),
            pl.BlockSpec((BR, D), lambda i: (i, 0)),
            pl.BlockSpec((BR, D), lambda i: (i, 0)),
            pl.BlockSpec((1, D), lambda i: (0, 0)),
        ],
        out_specs=pl.BlockSpec((BR, D), lambda i: (i, 0)),
        out_shape=jax.ShapeDtypeStruct((NPAD, D), jnp.float32),
    )(z, y, dinv, b)


def kernel(x, edge_index, W1, b1, W2, b2):
    ei = edge_index.astype(jnp.int32)
    src = jnp.concatenate([ei[0], jnp.zeros((EPAD - E,), jnp.int32)])
    dst = jnp.concatenate([ei[1], jnp.full((EPAD - E,), N, jnp.int32)])
    xp = jnp.concatenate([x, jnp.zeros((NPAD - N, D), jnp.float32)])

    hist = _deg_kernel(dst)
    y1, dinv = _mm1(hist, xp, W1)
    z1 = _edge_kernel(y1, src, dst)
    y2 = _mm2(z1, y1, dinv, b1.reshape(1, D), W2)
    z2 = _edge_kernel(y2, src, dst)
    out = _fin(z2, y2, dinv, b2.reshape(1, D))
    return out[:N]
